# baseline (device time: 85053 ns/iter reference)
import jax
import jax.numpy as jnp
from jax import lax
from jax.experimental import pallas as pl
from jax.experimental.pallas import tpu as pltpu

V_PER_SHARD = 8192


def kernel(ids, E):
    t = ids.shape[0]
    d = E.shape[1]

    my_y = lax.axis_index("y")
    base = my_y * V_PER_SHARD
    local = ids - base
    owned = (local >= 0) & (local < V_PER_SHARD)
    rows = jnp.where(owned, local, 0)
    partial = E[rows] * owned[:, None].astype(E.dtype)

    def body(p_ref, out_ref, comm_ref, send_sem, recv_sem):
        x = lax.axis_index("x")
        y = lax.axis_index("y")
        z = lax.axis_index("z")
        partner = (x, 1 - y, z)

        barrier = pltpu.get_barrier_semaphore()
        pl.semaphore_signal(
            barrier, inc=1, device_id=partner,
            device_id_type=pl.DeviceIdType.MESH,
        )
        pl.semaphore_wait(barrier, 1)

        rdma = pltpu.make_async_remote_copy(
            src_ref=p_ref,
            dst_ref=comm_ref,
            send_sem=send_sem,
            recv_sem=recv_sem,
            device_id=partner,
            device_id_type=pl.DeviceIdType.MESH,
        )
        rdma.start()
        rdma.wait()
        out_ref[...] = p_ref[...] + comm_ref[...]

    return pl.pallas_call(
        body,
        out_shape=jax.ShapeDtypeStruct((t, d), jnp.float32),
        in_specs=[pl.BlockSpec(memory_space=pltpu.VMEM)],
        out_specs=pl.BlockSpec(memory_space=pltpu.VMEM),
        scratch_shapes=[
            pltpu.VMEM((t, d), jnp.float32),
            pltpu.SemaphoreType.DMA,
            pltpu.SemaphoreType.DMA,
        ],
        compiler_params=pltpu.CompilerParams(collective_id=0),
    )(partial)


# device time: 45168 ns/iter; 1.8830x vs baseline; 1.8830x over previous
import jax
import jax.numpy as jnp
from jax import lax
from jax.experimental import pallas as pl
from jax.experimental.pallas import tpu as pltpu

V_PER_SHARD = 8192


def kernel(ids, E):
    t = ids.shape[0]
    d = E.shape[1]

    my_y = lax.axis_index("y")
    base = my_y * V_PER_SHARD
    local = ids - base
    owned = (local >= 0) & (local < V_PER_SHARD)
    n_mine = jnp.sum(owned.astype(jnp.int32))

    idx = jnp.cumsum(owned.astype(jnp.int32)) - 1
    pos = jnp.where(owned, idx, t)
    tok_mine = (
        jnp.zeros((t + 1,), jnp.int32)
        .at[pos]
        .set(jnp.arange(t, dtype=jnp.int32), mode="drop")[:t]
    )
    row_mine = jnp.where(owned, local, 0)[tok_mine]
    counts = jnp.stack([n_mine, t - n_mine]).astype(jnp.int32)

    def body(tok_ref, row_ref, cnt_ref, E_ref, out_ref, gsem, send_sem, recv_sem):
        x = lax.axis_index("x")
        y = lax.axis_index("y")
        z = lax.axis_index("z")
        partner = (x, 1 - y, z)
        nm = cnt_ref[0]
        no = cnt_ref[1]

        barrier = pltpu.get_barrier_semaphore()
        pl.semaphore_signal(
            barrier, inc=1, device_id=partner,
            device_id_type=pl.DeviceIdType.MESH,
        )
        pl.semaphore_wait(barrier, 1)

        def issue(j, carry):
            rid = row_ref[j]
            tk = tok_ref[j]
            src = E_ref.at[pl.ds(rid, 1)]
            dst = out_ref.at[pl.ds(tk, 1)]
            pltpu.make_async_copy(src, dst, gsem).start()
            pltpu.make_async_remote_copy(
                src_ref=src, dst_ref=dst,
                send_sem=send_sem, recv_sem=recv_sem,
                device_id=partner, device_id_type=pl.DeviceIdType.MESH,
            ).start()
            return carry

        lax.fori_loop(0, nm, issue, 0)

        def wait_local(j, carry):
            pltpu.make_async_copy(
                E_ref.at[pl.ds(0, 1)], out_ref.at[pl.ds(0, 1)], gsem
            ).wait()
            return carry

        def wait_send(j, carry):
            pltpu.make_async_remote_copy(
                src_ref=E_ref.at[pl.ds(0, 1)],
                dst_ref=out_ref.at[pl.ds(0, 1)],
                send_sem=send_sem, recv_sem=recv_sem,
                device_id=partner, device_id_type=pl.DeviceIdType.MESH,
            ).wait_send()
            return carry

        def wait_recv(j, carry):
            pltpu.make_async_remote_copy(
                src_ref=E_ref.at[pl.ds(0, 1)],
                dst_ref=out_ref.at[pl.ds(0, 1)],
                send_sem=send_sem, recv_sem=recv_sem,
                device_id=partner, device_id_type=pl.DeviceIdType.MESH,
            ).wait_recv()
            return carry

        lax.fori_loop(0, nm, wait_local, 0)
        lax.fori_loop(0, nm, wait_send, 0)
        lax.fori_loop(0, no, wait_recv, 0)

    return pl.pallas_call(
        body,
        out_shape=jax.ShapeDtypeStruct((t, d), jnp.float32),
        in_specs=[
            pl.BlockSpec(memory_space=pltpu.SMEM),
            pl.BlockSpec(memory_space=pltpu.SMEM),
            pl.BlockSpec(memory_space=pltpu.SMEM),
            pl.BlockSpec(memory_space=pltpu.MemorySpace.HBM),
        ],
        out_specs=pl.BlockSpec(memory_space=pltpu.VMEM),
        scratch_shapes=[
            pltpu.SemaphoreType.DMA,
            pltpu.SemaphoreType.DMA,
            pltpu.SemaphoreType.DMA,
        ],
        compiler_params=pltpu.CompilerParams(collective_id=0),
    )(tok_mine, row_mine, counts, E)


# device time: 44296 ns/iter; 1.9201x vs baseline; 1.0197x over previous
import jax
import jax.numpy as jnp
from jax import lax
from jax.experimental import pallas as pl
from jax.experimental.pallas import tpu as pltpu

V_PER_SHARD = 8192


def kernel(ids, E):
    t = ids.shape[0]
    d = E.shape[1]

    def body(ids_ref, E_ref, out_ref, gsem, send_sem, recv_sem):
        x = lax.axis_index("x")
        y = lax.axis_index("y")
        z = lax.axis_index("z")
        partner = (x, 1 - y, z)
        base = y * V_PER_SHARD

        barrier = pltpu.get_barrier_semaphore()
        pl.semaphore_signal(
            barrier, inc=1, device_id=partner,
            device_id_type=pl.DeviceIdType.MESH,
        )
        pl.semaphore_wait(barrier, 1)

        def issue(tk, nm):
            rid = ids_ref[tk] - base
            owned = jnp.logical_and(rid >= 0, rid < V_PER_SHARD)

            @pl.when(owned)
            def _():
                src = E_ref.at[pl.ds(rid, 1)]
                dst = out_ref.at[pl.ds(tk, 1)]
                pltpu.make_async_copy(src, dst, gsem).start()
                pltpu.make_async_remote_copy(
                    src_ref=src, dst_ref=dst,
                    send_sem=send_sem, recv_sem=recv_sem,
                    device_id=partner, device_id_type=pl.DeviceIdType.MESH,
                ).start()

            return nm + owned.astype(jnp.int32)

        nm = lax.fori_loop(0, t, issue, 0)
        no = t - nm

        def waits(j, carry):
            @pl.when(j < nm)
            def _():
                pltpu.make_async_copy(
                    E_ref.at[pl.ds(0, 1)], out_ref.at[pl.ds(0, 1)], gsem
                ).wait()
                pltpu.make_async_remote_copy(
                    src_ref=E_ref.at[pl.ds(0, 1)],
                    dst_ref=out_ref.at[pl.ds(0, 1)],
                    send_sem=send_sem, recv_sem=recv_sem,
                    device_id=partner, device_id_type=pl.DeviceIdType.MESH,
                ).wait_send()

            @pl.when(j < no)
            def _():
                pltpu.make_async_remote_copy(
                    src_ref=E_ref.at[pl.ds(0, 1)],
                    dst_ref=out_ref.at[pl.ds(0, 1)],
                    send_sem=send_sem, recv_sem=recv_sem,
                    device_id=partner, device_id_type=pl.DeviceIdType.MESH,
                ).wait_recv()

            return carry

        lax.fori_loop(0, t, waits, 0)

    return pl.pallas_call(
        body,
        out_shape=jax.ShapeDtypeStruct((t, d), jnp.float32),
        in_specs=[
            pl.BlockSpec(memory_space=pltpu.SMEM),
            pl.BlockSpec(memory_space=pltpu.MemorySpace.HBM),
        ],
        out_specs=pl.BlockSpec(memory_space=pltpu.VMEM),
        scratch_shapes=[
            pltpu.SemaphoreType.DMA,
            pltpu.SemaphoreType.DMA,
            pltpu.SemaphoreType.DMA,
        ],
        compiler_params=pltpu.CompilerParams(collective_id=0),
    )(ids, E)


# device time: 41529 ns/iter; 2.0480x vs baseline; 1.0666x over previous
import jax
import jax.numpy as jnp
from jax import lax
from jax.experimental import pallas as pl
from jax.experimental.pallas import tpu as pltpu

V_PER_SHARD = 8192


def kernel(ids, E):
    t = ids.shape[0]
    d = E.shape[1]

    def body(ids_ref, E_ref, out_ref, gsem, send_sem, recv_sem):
        x = lax.axis_index("x")
        y = lax.axis_index("y")
        z = lax.axis_index("z")
        partner = x * 8 + (1 - y) * 4 + z
        base = y * V_PER_SHARD

        barrier = pltpu.get_barrier_semaphore()
        pl.semaphore_signal(
            barrier, inc=1, device_id=partner,
            device_id_type=pl.DeviceIdType.LOGICAL,
        )
        pl.semaphore_wait(barrier, 1)

        def issue(tk, nm):
            rid = ids_ref[tk] - base
            owned = jnp.logical_and(rid >= 0, rid < V_PER_SHARD)

            @pl.when(owned)
            def _():
                src = E_ref.at[pl.ds(rid, 1)]
                dst = out_ref.at[pl.ds(tk, 1)]
                pltpu.make_async_copy(src, dst, gsem).start()
                pltpu.make_async_remote_copy(
                    src_ref=src, dst_ref=dst,
                    send_sem=send_sem, recv_sem=recv_sem,
                    device_id=partner, device_id_type=pl.DeviceIdType.LOGICAL,
                ).start()

            return nm + owned.astype(jnp.int32)

        nm = lax.fori_loop(0, t, issue, 0, unroll=4)
        no = t - nm

        src0 = E_ref.at[pl.ds(0, 1)]
        dst0 = out_ref.at[pl.ds(0, 1)]

        def waits(j, carry):
            @pl.when(j < nm)
            def _():
                pltpu.make_async_copy(src0, dst0, gsem).wait()
                pltpu.make_async_remote_copy(
                    src_ref=src0, dst_ref=dst0,
                    send_sem=send_sem, recv_sem=recv_sem,
                    device_id=partner, device_id_type=pl.DeviceIdType.LOGICAL,
                ).wait_send()

            @pl.when(j < no)
            def _():
                pltpu.make_async_remote_copy(
                    src_ref=src0, dst_ref=dst0,
                    send_sem=send_sem, recv_sem=recv_sem,
                    device_id=partner, device_id_type=pl.DeviceIdType.LOGICAL,
                ).wait_recv()

            return carry

        lax.fori_loop(0, t, waits, 0, unroll=8)

    return pl.pallas_call(
        body,
        out_shape=jax.ShapeDtypeStruct((t, d), jnp.float32),
        in_specs=[
            pl.BlockSpec(memory_space=pltpu.SMEM),
            pl.BlockSpec(memory_space=pltpu.MemorySpace.HBM),
        ],
        out_specs=pl.BlockSpec(memory_space=pltpu.VMEM),
        scratch_shapes=[
            pltpu.SemaphoreType.DMA,
            pltpu.SemaphoreType.DMA,
            pltpu.SemaphoreType.DMA,
        ],
        compiler_params=pltpu.CompilerParams(collective_id=0),
    )(ids, E)


# device time: 38903 ns/iter; 2.1863x vs baseline; 1.0675x over previous
import jax
import jax.numpy as jnp
from jax import lax
from jax.experimental import pallas as pl
from jax.experimental.pallas import tpu as pltpu

V_PER_SHARD = 8192


def kernel(ids, E):
    t = ids.shape[0]
    d = E.shape[1]

    def body(ids_ref, E_ref, out_ref, gsem, send_sem, recv_sem):
        x = lax.axis_index("x")
        y = lax.axis_index("y")
        z = lax.axis_index("z")
        partner = x * 8 + (1 - y) * 4 + z
        base = y * V_PER_SHARD

        barrier = pltpu.get_barrier_semaphore()
        pl.semaphore_signal(
            barrier, inc=1, device_id=partner,
            device_id_type=pl.DeviceIdType.LOGICAL,
        )
        pl.semaphore_wait(barrier, 1)

        def issue(tk, nm):
            rid = ids_ref[tk] - base
            owned = rid.astype(jnp.uint32) < V_PER_SHARD

            @pl.when(owned)
            def _():
                src = E_ref.at[pl.ds(rid, 1)]
                dst = out_ref.at[pl.ds(tk, 1)]
                pltpu.make_async_copy(src, dst, gsem).start()
                pltpu.make_async_remote_copy(
                    src_ref=src, dst_ref=dst,
                    send_sem=send_sem, recv_sem=recv_sem,
                    device_id=partner, device_id_type=pl.DeviceIdType.LOGICAL,
                ).start()

            return nm + owned.astype(jnp.int32)

        nm = lax.fori_loop(0, t, issue, 0, unroll=8)
        no = t - nm

        src0 = E_ref.at[pl.ds(0, 1)]
        dst0 = out_ref.at[pl.ds(0, 1)]

        def waits(j, carry):
            @pl.when(j < nm)
            def _():
                pltpu.make_async_copy(src0, dst0, gsem).wait()
                pltpu.make_async_remote_copy(
                    src_ref=src0, dst_ref=dst0,
                    send_sem=send_sem, recv_sem=recv_sem,
                    device_id=partner, device_id_type=pl.DeviceIdType.LOGICAL,
                ).wait_send()

            @pl.when(j < no)
            def _():
                pltpu.make_async_remote_copy(
                    src_ref=src0, dst_ref=dst0,
                    send_sem=send_sem, recv_sem=recv_sem,
                    device_id=partner, device_id_type=pl.DeviceIdType.LOGICAL,
                ).wait_recv()

            return carry

        lax.fori_loop(0, t, waits, 0, unroll=8)

    return pl.pallas_call(
        body,
        out_shape=jax.ShapeDtypeStruct((t, d), jnp.float32),
        in_specs=[
            pl.BlockSpec(memory_space=pltpu.SMEM),
            pl.BlockSpec(memory_space=pltpu.MemorySpace.HBM),
        ],
        out_specs=pl.BlockSpec(memory_space=pltpu.VMEM),
        scratch_shapes=[
            pltpu.SemaphoreType.DMA,
            pltpu.SemaphoreType.DMA,
            pltpu.SemaphoreType.DMA,
        ],
        compiler_params=pltpu.CompilerParams(collective_id=0),
    )(ids, E)


# device time: 32969 ns/iter; 2.5798x vs baseline; 1.1800x over previous
import jax
import jax.numpy as jnp
from jax import lax
from jax.experimental import pallas as pl
from jax.experimental.pallas import tpu as pltpu

V_PER_SHARD = 8192


def kernel(ids, E):
    t = ids.shape[0]
    d = E.shape[1]

    def body(ids_ref, E_ref, out_ref, tok_ref, row_ref, gsem, send_sem, recv_sem):
        x = lax.axis_index("x")
        y = lax.axis_index("y")
        z = lax.axis_index("z")
        partner = x * 8 + (1 - y) * 4 + z
        base = y * V_PER_SHARD

        barrier = pltpu.get_barrier_semaphore()
        pl.semaphore_signal(
            barrier, inc=1, device_id=partner,
            device_id_type=pl.DeviceIdType.LOGICAL,
        )
        pl.semaphore_wait(barrier, 1)

        def scan(tk, nm):
            rid = ids_ref[tk] - base
            owned = rid.astype(jnp.uint32) < V_PER_SHARD

            @pl.when(owned)
            def _():
                tok_ref[nm] = tk
                row_ref[nm] = rid

            return nm + owned.astype(jnp.int32)

        nm = lax.fori_loop(0, t, scan, 0, unroll=8)
        no = t - nm

        def issue(j, carry):
            rid = row_ref[j]
            tk = tok_ref[j]
            src = E_ref.at[pl.ds(rid, 1)]
            dst = out_ref.at[pl.ds(tk, 1)]
            pltpu.make_async_copy(src, dst, gsem).start()
            pltpu.make_async_remote_copy(
                src_ref=src, dst_ref=dst,
                send_sem=send_sem, recv_sem=recv_sem,
                device_id=partner, device_id_type=pl.DeviceIdType.LOGICAL,
            ).start()
            return carry

        lax.fori_loop(0, nm, issue, 0)

        src0 = E_ref.at[pl.ds(0, 1)]
        dst0 = out_ref.at[pl.ds(0, 1)]

        def waits_mine(j, carry):
            pltpu.make_async_copy(src0, dst0, gsem).wait()
            pltpu.make_async_remote_copy(
                src_ref=src0, dst_ref=dst0,
                send_sem=send_sem, recv_sem=recv_sem,
                device_id=partner, device_id_type=pl.DeviceIdType.LOGICAL,
            ).wait_send()
            return carry

        def waits_recv(j, carry):
            pltpu.make_async_remote_copy(
                src_ref=src0, dst_ref=dst0,
                send_sem=send_sem, recv_sem=recv_sem,
                device_id=partner, device_id_type=pl.DeviceIdType.LOGICAL,
            ).wait_recv()
            return carry

        lax.fori_loop(0, nm, waits_mine, 0)
        lax.fori_loop(0, no, waits_recv, 0)

    return pl.pallas_call(
        body,
        out_shape=jax.ShapeDtypeStruct((t, d), jnp.float32),
        in_specs=[
            pl.BlockSpec(memory_space=pltpu.SMEM),
            pl.BlockSpec(memory_space=pltpu.MemorySpace.HBM),
        ],
        out_specs=pl.BlockSpec(memory_space=pltpu.VMEM),
        scratch_shapes=[
            pltpu.SMEM((t,), jnp.int32),
            pltpu.SMEM((t,), jnp.int32),
            pltpu.SemaphoreType.DMA,
            pltpu.SemaphoreType.DMA,
            pltpu.SemaphoreType.DMA,
        ],
        compiler_params=pltpu.CompilerParams(collective_id=0),
    )(ids, E)
